# baseline (speedup 1.0000x reference)
import jax
import jax.numpy as jnp
from jax import lax
from jax.experimental import pallas as pl
from jax.experimental.pallas import tpu as pltpu
from jax.experimental.pallas import tpu_sc as plsc

_B = 4
_S = 4096
_D = 1024
_S_TC = 3584
_S_SC = _S - _S_TC
_BS = 512

_NC = 2
_NS = 16
_NW = _NC * _NS
_SW = _S_SC // _NW
_R = min(16, _SW)
_NCHUNK = _SW // _R
_NT = _NCHUNK * _B
_L = 16
_U = 8


def _tc_body(x_ref, pe_ref, o_ref):
    o_ref[...] = x_ref[...] + pe_ref[...]


def _sc_body(x_hbm, pe_hbm, o_hbm,
             pe0, pe1, xb0, xb1, xb2,
             spe0, spe1, si0, si1, si2, so0, so1, so2):
    wid = lax.axis_index("s") * _NC + lax.axis_index("c")
    l0 = wid * _SW
    s0 = _S_TC + l0
    pebufs, spe = (pe0, pe1), (spe0, spe1)
    xbufs, sin, sout = (xb0, xb1, xb2), (si0, si1, si2), (so0, so1, so2)

    def pe_copy(ci):
        return pltpu.make_async_copy(
            pe_hbm.at[pl.ds(s0 + ci * _R, _R)], pebufs[ci % 2], spe[ci % 2])

    def in_copy(t):
        ci, b = t // _B, t % _B
        p = t % 3
        return pltpu.make_async_copy(
            x_hbm.at[b, pl.ds(s0 + ci * _R, _R)], xbufs[p], sin[p])

    def out_copy(t):
        ci, b = t // _B, t % _B
        p = t % 3
        return pltpu.make_async_copy(
            xbufs[p], o_hbm.at[b, pl.ds(l0 + ci * _R, _R)], sout[p])

    pe_copy(0).start()
    in_copy(0).start()
    in_copy(1).start()

    for t in range(_NT):
        ci, b = t // _B, t % _B
        if b == 0:
            pe_copy(ci).wait()
            if ci + 1 < _NCHUNK:
                pe_copy(ci + 1).start()
        in_copy(t).wait()
        xbuf, pebuf = xbufs[t % 3], pebufs[ci % 2]

        def row(r, carry):
            def col(i, c3):
                for u in range(_U):
                    c = (i * _U + u) * _L
                    plsc.addupdate(xbuf.at[r, pl.ds(c, _L)],
                                   pebuf[r, pl.ds(c, _L)])
                return c3
            return lax.fori_loop(0, _D // (_L * _U), col, carry)

        lax.fori_loop(0, _R, row, 0)
        out_copy(t).start()
        if t + 2 < _NT:
            if t >= 1:
                out_copy(t - 1).wait()
            in_copy(t + 2).start()
    out_copy(_NT - 3).wait()
    out_copy(_NT - 2).wait()
    out_copy(_NT - 1).wait()


@jax.jit
def _hybrid_add(x, pe):
    tc_out = pl.pallas_call(
        _tc_body,
        grid=(_S_TC // _BS,),
        in_specs=[
            pl.BlockSpec((_B, _BS, _D), lambda s: (0, s, 0)),
            pl.BlockSpec((_BS, _D), lambda s: (s, 0)),
        ],
        out_specs=pl.BlockSpec((_B, _BS, _D), lambda s: (0, s, 0)),
        out_shape=jax.ShapeDtypeStruct((_B, _S, _D), jnp.float32),
        cost_estimate=pl.CostEstimate(
            flops=_B * _S_TC * _D,
            transcendentals=0,
            bytes_accessed=(2 * _B * _S_TC * _D + _S_TC * _D) * 4,
        ),
    )(x, pe)

    sc_out = pl.kernel(
        _sc_body,
        out_type=jax.ShapeDtypeStruct((_B, _S_SC, _D), jnp.float32),
        mesh=plsc.VectorSubcoreMesh(core_axis_name="c", subcore_axis_name="s"),
        scratch_types=[
            pltpu.VMEM((_R, _D), jnp.float32),
            pltpu.VMEM((_R, _D), jnp.float32),
            pltpu.VMEM((_R, _D), jnp.float32),
            pltpu.VMEM((_R, _D), jnp.float32),
            pltpu.VMEM((_R, _D), jnp.float32),
            pltpu.SemaphoreType.DMA,
            pltpu.SemaphoreType.DMA,
            pltpu.SemaphoreType.DMA,
            pltpu.SemaphoreType.DMA,
            pltpu.SemaphoreType.DMA,
            pltpu.SemaphoreType.DMA,
            pltpu.SemaphoreType.DMA,
            pltpu.SemaphoreType.DMA,
        ],
        cost_estimate=pl.CostEstimate(
            flops=_B * _S_SC * _D,
            transcendentals=0,
            bytes_accessed=(2 * _B * _S_SC * _D + _S_SC * _D) * 4,
        ),
    )(x, pe)

    return lax.dynamic_update_slice(tc_out, sc_out, (0, _S_TC, 0))


def kernel(x, pos_embed):
    return _hybrid_add(x, pos_embed[:_S])


# speedup vs baseline: 1.0820x; 1.0820x over previous
import jax
import jax.numpy as jnp
from jax import lax
from jax.experimental import pallas as pl
from jax.experimental.pallas import tpu as pltpu
from jax.experimental.pallas import tpu_sc as plsc

_B = 4
_S = 4096
_D = 1024
_S_TC = 3840
_S_SC = _S - _S_TC
_BS = 512

_NC = 2
_NS = 16
_NW = _NC * _NS
_SW = _S_SC // _NW
_R = min(16, _SW)
_NCHUNK = _SW // _R
_NT = _NCHUNK * _B
_L = 16
_U = 8


def _tc_body(x_ref, pe_ref, o_ref):
    o_ref[...] = x_ref[...] + pe_ref[...]


def _sc_body(x_hbm, pe_hbm, o_hbm,
             pe0, pe1, xb0, xb1, xb2,
             spe0, spe1, si0, si1, si2, so0, so1, so2):
    wid = lax.axis_index("s") * _NC + lax.axis_index("c")
    l0 = wid * _SW
    s0 = _S_TC + l0
    pebufs, spe = (pe0, pe1), (spe0, spe1)
    xbufs, sin, sout = (xb0, xb1, xb2), (si0, si1, si2), (so0, so1, so2)

    def pe_copy(ci):
        return pltpu.make_async_copy(
            pe_hbm.at[pl.ds(s0 + ci * _R, _R)], pebufs[ci % 2], spe[ci % 2])

    def in_copy(t):
        ci, b = t // _B, t % _B
        p = t % 3
        return pltpu.make_async_copy(
            x_hbm.at[b, pl.ds(s0 + ci * _R, _R)], xbufs[p], sin[p])

    def out_copy(t):
        ci, b = t // _B, t % _B
        p = t % 3
        return pltpu.make_async_copy(
            xbufs[p], o_hbm.at[b, pl.ds(l0 + ci * _R, _R)], sout[p])

    pe_copy(0).start()
    in_copy(0).start()
    in_copy(1).start()

    for t in range(_NT):
        ci, b = t // _B, t % _B
        if b == 0:
            pe_copy(ci).wait()
            if ci + 1 < _NCHUNK:
                pe_copy(ci + 1).start()
        in_copy(t).wait()
        xbuf, pebuf = xbufs[t % 3], pebufs[ci % 2]

        def row(r, carry):
            def col(i, c3):
                for u in range(_U):
                    c = (i * _U + u) * _L
                    plsc.addupdate(xbuf.at[r, pl.ds(c, _L)],
                                   pebuf[r, pl.ds(c, _L)])
                return c3
            return lax.fori_loop(0, _D // (_L * _U), col, carry)

        lax.fori_loop(0, _R, row, 0)
        out_copy(t).start()
        if t + 2 < _NT:
            if t >= 1:
                out_copy(t - 1).wait()
            in_copy(t + 2).start()
    out_copy(_NT - 3).wait()
    out_copy(_NT - 2).wait()
    out_copy(_NT - 1).wait()


@jax.jit
def _hybrid_add(x, pe):
    tc_out = pl.pallas_call(
        _tc_body,
        grid=(_S_TC // _BS,),
        in_specs=[
            pl.BlockSpec((_B, _BS, _D), lambda s: (0, s, 0)),
            pl.BlockSpec((_BS, _D), lambda s: (s, 0)),
        ],
        out_specs=pl.BlockSpec((_B, _BS, _D), lambda s: (0, s, 0)),
        out_shape=jax.ShapeDtypeStruct((_B, _S, _D), jnp.float32),
        cost_estimate=pl.CostEstimate(
            flops=_B * _S_TC * _D,
            transcendentals=0,
            bytes_accessed=(2 * _B * _S_TC * _D + _S_TC * _D) * 4,
        ),
    )(x, pe)

    sc_out = pl.kernel(
        _sc_body,
        out_type=jax.ShapeDtypeStruct((_B, _S_SC, _D), jnp.float32),
        mesh=plsc.VectorSubcoreMesh(core_axis_name="c", subcore_axis_name="s"),
        scratch_types=[
            pltpu.VMEM((_R, _D), jnp.float32),
            pltpu.VMEM((_R, _D), jnp.float32),
            pltpu.VMEM((_R, _D), jnp.float32),
            pltpu.VMEM((_R, _D), jnp.float32),
            pltpu.VMEM((_R, _D), jnp.float32),
            pltpu.SemaphoreType.DMA,
            pltpu.SemaphoreType.DMA,
            pltpu.SemaphoreType.DMA,
            pltpu.SemaphoreType.DMA,
            pltpu.SemaphoreType.DMA,
            pltpu.SemaphoreType.DMA,
            pltpu.SemaphoreType.DMA,
            pltpu.SemaphoreType.DMA,
        ],
        cost_estimate=pl.CostEstimate(
            flops=_B * _S_SC * _D,
            transcendentals=0,
            bytes_accessed=(2 * _B * _S_SC * _D + _S_SC * _D) * 4,
        ),
    )(x, pe)

    return lax.dynamic_update_slice(tc_out, sc_out, (0, _S_TC, 0))


def kernel(x, pos_embed):
    return _hybrid_add(x, pos_embed[:_S])
